# Initial kernel scaffold; baseline (speedup 1.0000x reference)
import jax
import jax.numpy as jnp
from jax.experimental import pallas as pl


def kernel(X, I):
    raise NotImplementedError("write your pallas kernel here")


# speedup vs baseline: 2.7596x; 2.7596x over previous
import functools

import jax
import jax.numpy as jnp
from jax import lax
from jax.experimental import pallas as pl
from jax.experimental.pallas import tpu as pltpu
from jax.experimental.pallas import tpu_sc as plsc

N_ROWS = 100000
N_FEAT = 256
N_SEG = 256
LANES = 16
VREGS = N_FEAT // LANES

NC = 2
NS = 16
NW = NC * NS
SEG_PER_W = N_SEG // NW

CHUNK = 128


def _sc_body(x_hbm, bounds_hbm, out_hbm, bvmem, buf, ovmem):
    wid = lax.axis_index("s") * NC + lax.axis_index("c")
    pltpu.sync_copy(bounds_hbm, bvmem)
    bvec = bvmem[pl.ds(wid * SEG_PER_W, LANES)]

    for j in range(SEG_PER_W):
        start = bvec[j]
        end = bvec[j + 1]
        acc0 = [jnp.zeros((LANES,), jnp.float32) for _ in range(VREGS)]
        a0 = (start // 8) * 8
        nch = lax.div(end - a0 + (CHUNK - 1), CHUNK)

        def chunk_body(c, acc, a0=a0, start=start, end=end):
            base = a0 + c * CHUNK
            base_c = jnp.minimum(base, N_ROWS - CHUNK)
            delta = base - base_c
            pltpu.sync_copy(x_hbm.at[pl.ds(base_c, CHUNK), :], buf)
            lo = jnp.maximum(0, start - base)
            hi = jnp.minimum(CHUNK, end - base)

            def row_body(r, a):
                rr = r + delta
                return [a[f] + buf[rr, pl.ds(f * LANES, LANES)]
                        for f in range(VREGS)]

            return lax.fori_loop(lo, hi, row_body, acc)

        acc = lax.fori_loop(0, nch, chunk_body, acc0)
        for f in range(VREGS):
            ovmem[j, pl.ds(f * LANES, LANES)] = acc[f]

    pltpu.sync_copy(ovmem, out_hbm.at[pl.ds(wid * SEG_PER_W, SEG_PER_W), :])


@jax.jit
def kernel(X, I):
    I32 = I.astype(jnp.int32)
    bounds = jnp.searchsorted(
        I32, jnp.arange(N_SEG + 1, dtype=jnp.int32), side="left"
    ).astype(jnp.int32)
    bounds = jnp.concatenate(
        [bounds, jnp.zeros((7,), jnp.int32)])

    mesh = plsc.VectorSubcoreMesh(
        core_axis_name="c", subcore_axis_name="s", num_cores=NC,
        num_subcores=NS)
    f = pl.kernel(
        _sc_body,
        out_type=jax.ShapeDtypeStruct((N_SEG, N_FEAT), jnp.float32),
        mesh=mesh,
        scratch_types=[
            pltpu.VMEM((N_SEG + 8,), jnp.int32),
            pltpu.VMEM((CHUNK, N_FEAT), jnp.float32),
            pltpu.VMEM((SEG_PER_W, N_FEAT), jnp.float32),
        ],
    )
    return f(X, bounds)


# speedup vs baseline: 3.7078x; 1.3436x over previous
import jax
import jax.numpy as jnp
from jax import lax
from jax.experimental import pallas as pl
from jax.experimental.pallas import tpu as pltpu
from jax.experimental.pallas import tpu_sc as plsc

N_ROWS = 100000
N_FEAT = 256
N_SEG = 256
LANES = 16
VREGS = N_FEAT // LANES

NC = 2
NS = 16
NW = NC * NS
SEG_PER_W = N_SEG // NW

CHUNK = 240


def _sc_body(x_hbm, bounds_hbm, out_hbm, bvmem, buf0, buf1, ovmem,
             sem0, sem1):
    wid = lax.axis_index("s") * NC + lax.axis_index("c")
    pltpu.sync_copy(bounds_hbm, bvmem)
    bvec = bvmem[pl.ds(wid * SEG_PER_W, LANES)]

    zero = jnp.zeros((LANES,), jnp.float32)
    for j in range(SEG_PER_W):
        for f in range(VREGS):
            ovmem[j, pl.ds(f * LANES, LANES)] = zero

    rs = bvec[0]
    re = bvec[SEG_PER_W]
    a0 = (rs // 8) * 8
    nch = lax.div(re - a0 + (CHUNK - 1), CHUNK)
    bufs = (buf0, buf1)
    sems = (sem0, sem1)

    def chunk_slice(c):
        base = a0 + c * CHUNK
        base_c = jnp.minimum(base, N_ROWS - CHUNK)
        return base, base_c

    def start_dma(c, par):
        _, base_c = chunk_slice(c)
        pltpu.async_copy(x_hbm.at[pl.ds(base_c, CHUNK), :], bufs[par],
                         sems[par])

    def wait_dma(c, par):
        _, base_c = chunk_slice(c)
        pltpu.make_async_copy(x_hbm.at[pl.ds(base_c, CHUNK), :], bufs[par],
                              sems[par]).wait()

    @pl.when(nch > 0)
    def _():
        start_dma(0, 0)

    def process(c, par):
        base, base_c = chunk_slice(c)
        delta = base - base_c
        buf = bufs[par]
        for j in range(SEG_PER_W):
            lo = jnp.maximum(bvec[j] - base, 0)
            hi = jnp.minimum(bvec[j + 1] - base, CHUNK)

            @pl.when(hi > lo)
            def _(j=j, lo=lo, hi=hi, buf=buf, delta=delta):
                acc0 = [ovmem[j, pl.ds(f * LANES, LANES)]
                        for f in range(VREGS)]

                def row_body(r, a):
                    rr = r + delta
                    return [a[f] + buf[rr, pl.ds(f * LANES, LANES)]
                            for f in range(VREGS)]

                acc = lax.fori_loop(lo, hi, row_body, acc0)
                for f in range(VREGS):
                    ovmem[j, pl.ds(f * LANES, LANES)] = acc[f]

    npairs = lax.div(nch + 1, 2)

    def pair_body(g, carry):
        for par in (0, 1):
            c = 2 * g + par

            @pl.when(c < nch)
            def _(c=c, par=par):
                wait_dma(c, par)

                @pl.when(c + 1 < nch)
                def _(c=c, par=par):
                    start_dma(c + 1, 1 - par)

                process(c, par)
        return carry

    lax.fori_loop(0, npairs, pair_body, 0)
    pltpu.sync_copy(ovmem, out_hbm.at[pl.ds(wid * SEG_PER_W, SEG_PER_W), :])


@jax.jit
def kernel(X, I):
    I32 = I.astype(jnp.int32)
    bounds = jnp.searchsorted(
        I32, jnp.arange(N_SEG + 1, dtype=jnp.int32), side="left"
    ).astype(jnp.int32)
    bounds = jnp.concatenate(
        [bounds, jnp.zeros((7,), jnp.int32)])

    mesh = plsc.VectorSubcoreMesh(
        core_axis_name="c", subcore_axis_name="s", num_cores=NC,
        num_subcores=NS)
    f = pl.kernel(
        _sc_body,
        out_type=jax.ShapeDtypeStruct((N_SEG, N_FEAT), jnp.float32),
        mesh=mesh,
        scratch_types=[
            pltpu.VMEM((N_SEG + 8,), jnp.int32),
            pltpu.VMEM((CHUNK, N_FEAT), jnp.float32),
            pltpu.VMEM((CHUNK, N_FEAT), jnp.float32),
            pltpu.VMEM((SEG_PER_W, N_FEAT), jnp.float32),
            pltpu.SemaphoreType.DMA,
            pltpu.SemaphoreType.DMA,
        ],
    )
    return f(X, bounds)


# speedup vs baseline: 4.6085x; 1.2429x over previous
import jax
import jax.numpy as jnp
from jax import lax
from jax.experimental import pallas as pl
from jax.experimental.pallas import tpu as pltpu
from jax.experimental.pallas import tpu_sc as plsc

N_ROWS = 100000
N_FEAT = 256
N_SEG = 256
LANES = 16
VREGS = N_FEAT // LANES

NC = 2
NS = 16
NW = NC * NS
SEG_PER_W = N_SEG // NW

CHUNK = 240


def _sc_body(x_hbm, bounds_hbm, out_hbm, bvmem, buf0, buf1, ovmem,
             sem0, sem1):
    wid = lax.axis_index("s") * NC + lax.axis_index("c")
    pltpu.sync_copy(bounds_hbm, bvmem)
    bvec = bvmem[pl.ds(wid * SEG_PER_W, LANES)]

    zero = jnp.zeros((LANES,), jnp.float32)
    for j in range(SEG_PER_W):
        for f in range(VREGS):
            ovmem[j, pl.ds(f * LANES, LANES)] = zero

    rs = bvec[0]
    re = bvec[SEG_PER_W]
    a0 = (rs // 8) * 8
    nch = lax.div(re - a0 + (CHUNK - 1), CHUNK)
    bufs = (buf0, buf1)
    sems = (sem0, sem1)

    def chunk_slice(c):
        base = a0 + c * CHUNK
        base_c = jnp.minimum(base, N_ROWS - CHUNK)
        return base, base_c

    def start_dma(c, par):
        _, base_c = chunk_slice(c)
        pltpu.async_copy(x_hbm.at[pl.ds(base_c, CHUNK), :], bufs[par],
                         sems[par])

    def wait_dma(c, par):
        _, base_c = chunk_slice(c)
        pltpu.make_async_copy(x_hbm.at[pl.ds(base_c, CHUNK), :], bufs[par],
                              sems[par]).wait()

    @pl.when(nch > 0)
    def _():
        start_dma(0, 0)

    def process(c, par):
        base, base_c = chunk_slice(c)
        delta = base - base_c
        buf = bufs[par]
        for j in range(SEG_PER_W):
            lo = jnp.maximum(bvec[j] - base, 0)
            hi = jnp.minimum(bvec[j + 1] - base, CHUNK)

            @pl.when(hi > lo)
            def _(j=j, lo=lo, hi=hi, buf=buf, delta=delta):
                acc0 = [ovmem[j, pl.ds(f * LANES, LANES)]
                        for f in range(VREGS)]

                def row_body(r, a):
                    rr = r + delta
                    return [a[f] + buf[rr, pl.ds(f * LANES, LANES)]
                            for f in range(VREGS)]

                acc = lax.fori_loop(lo, hi, row_body, acc0)
                for f in range(VREGS):
                    ovmem[j, pl.ds(f * LANES, LANES)] = acc[f]

    npairs = lax.div(nch + 1, 2)

    def pair_body(g, carry):
        for par in (0, 1):
            c = 2 * g + par

            @pl.when(c < nch)
            def _(c=c, par=par):
                wait_dma(c, par)

                @pl.when(c + 1 < nch)
                def _(c=c, par=par):
                    start_dma(c + 1, 1 - par)

                process(c, par)
        return carry

    lax.fori_loop(0, npairs, pair_body, 0)
    pltpu.sync_copy(ovmem, out_hbm.at[pl.ds(wid * SEG_PER_W, SEG_PER_W), :])


@jax.jit
def kernel(X, I):
    I32 = I.astype(jnp.int32)
    bounds = jnp.searchsorted(
        I32, jnp.arange(N_SEG + 1, dtype=jnp.int32), side="left",
        method="scan_unrolled").astype(jnp.int32)
    bounds = jnp.concatenate(
        [bounds, jnp.zeros((7,), jnp.int32)])

    mesh = plsc.VectorSubcoreMesh(
        core_axis_name="c", subcore_axis_name="s", num_cores=NC,
        num_subcores=NS)
    f = pl.kernel(
        _sc_body,
        out_type=jax.ShapeDtypeStruct((N_SEG, N_FEAT), jnp.float32),
        mesh=mesh,
        scratch_types=[
            pltpu.VMEM((N_SEG + 8,), jnp.int32),
            pltpu.VMEM((CHUNK, N_FEAT), jnp.float32),
            pltpu.VMEM((CHUNK, N_FEAT), jnp.float32),
            pltpu.VMEM((SEG_PER_W, N_FEAT), jnp.float32),
            pltpu.SemaphoreType.DMA,
            pltpu.SemaphoreType.DMA,
        ],
    )
    return f(X, bounds)


# speedup vs baseline: 6.1129x; 1.3265x over previous
import jax
import jax.numpy as jnp
from jax import lax
from jax.experimental import pallas as pl
from jax.experimental.pallas import tpu as pltpu
from jax.experimental.pallas import tpu_sc as plsc

N_ROWS = 100000
N_FEAT = 256
N_SEG = 256
LANES = 16
VREGS = N_FEAT // LANES
N_TROW = N_ROWS // LANES

NC = 2
NS = 16
NW = NC * NS
SEG_PER_W = N_SEG // NW

CHUNK = 240


def _sc_body(x_hbm, ids_hbm, out_hbm, probe, buf0, buf1, ovmem,
             semp, sem0, sem1):
    wid = lax.axis_index("s") * NC + lax.axis_index("c")
    nb = SEG_PER_W + 1

    def probe_rows(rows):
        for j in range(nb):
            pltpu.async_copy(ids_hbm.at[pl.ds(rows[j] * LANES, LANES)],
                             probe.at[j, :], semp)
        for j in range(nb):
            pltpu.make_async_copy(
                ids_hbm.at[pl.ds(rows[j] * LANES, LANES)],
                probe.at[j, :], semp).wait()

    los = [jnp.int32(0)] * nb
    his = [jnp.int32(N_TROW)] * nb
    for _ in range(13):
        mids = [(los[j] + his[j]) >> 1 for j in range(nb)]
        probe_rows(mids)
        for j in range(nb):
            pred = probe[j, :][0] < (wid * SEG_PER_W + j)
            los[j] = jnp.where(pred, mids[j] + 1, los[j])
            his[j] = jnp.where(pred, his[j], mids[j])
    r0s = [jnp.maximum(los[j] - 1, 0) for j in range(nb)]
    probe_rows(r0s)
    b = []
    for j in range(nb):
        row = probe[j, :]
        s = wid * SEG_PER_W + j
        cnt = jnp.int32(0)
        for k in range(LANES):
            cnt = cnt + jnp.where(row[k] < s, jnp.int32(1), jnp.int32(0))
        b.append(r0s[j] * LANES + cnt)

    zero = jnp.zeros((LANES,), jnp.float32)
    for j in range(SEG_PER_W):
        for f in range(VREGS):
            ovmem[j, pl.ds(f * LANES, LANES)] = zero

    rs = b[0]
    re = b[SEG_PER_W]
    a0 = (rs // 8) * 8
    nch = lax.div(re - a0 + (CHUNK - 1), CHUNK)
    bufs = (buf0, buf1)
    sems = (sem0, sem1)

    def chunk_slice(c):
        base = a0 + c * CHUNK
        base_c = jnp.minimum(base, N_ROWS - CHUNK)
        return base, base_c

    def start_dma(c, par):
        _, base_c = chunk_slice(c)
        pltpu.async_copy(x_hbm.at[pl.ds(base_c, CHUNK), :], bufs[par],
                         sems[par])

    def wait_dma(c, par):
        _, base_c = chunk_slice(c)
        pltpu.make_async_copy(x_hbm.at[pl.ds(base_c, CHUNK), :], bufs[par],
                              sems[par]).wait()

    @pl.when(nch > 0)
    def _():
        start_dma(0, 0)

    def process(c, par):
        base, base_c = chunk_slice(c)
        delta = base - base_c
        buf = bufs[par]
        for j in range(SEG_PER_W):
            lo = jnp.maximum(b[j] - base, 0)
            hi = jnp.minimum(b[j + 1] - base, CHUNK)

            @pl.when(hi > lo)
            def _(j=j, lo=lo, hi=hi, buf=buf, delta=delta):
                acc0 = [ovmem[j, pl.ds(f * LANES, LANES)]
                        for f in range(VREGS)]

                def row_body(r, a):
                    rr = r + delta
                    return [a[f] + buf[rr, pl.ds(f * LANES, LANES)]
                            for f in range(VREGS)]

                acc = lax.fori_loop(lo, hi, row_body, acc0)
                for f in range(VREGS):
                    ovmem[j, pl.ds(f * LANES, LANES)] = acc[f]

    npairs = lax.div(nch + 1, 2)

    def pair_body(g, carry):
        for par in (0, 1):
            c = 2 * g + par

            @pl.when(c < nch)
            def _(c=c, par=par):
                wait_dma(c, par)

                @pl.when(c + 1 < nch)
                def _(c=c, par=par):
                    start_dma(c + 1, 1 - par)

                process(c, par)
        return carry

    lax.fori_loop(0, npairs, pair_body, 0)
    pltpu.sync_copy(ovmem, out_hbm.at[pl.ds(wid * SEG_PER_W, SEG_PER_W), :])


@jax.jit
def kernel(X, I):
    ids = I.astype(jnp.int32)
    mesh = plsc.VectorSubcoreMesh(
        core_axis_name="c", subcore_axis_name="s", num_cores=NC,
        num_subcores=NS)
    f = pl.kernel(
        _sc_body,
        out_type=jax.ShapeDtypeStruct((N_SEG, N_FEAT), jnp.float32),
        mesh=mesh,
        scratch_types=[
            pltpu.VMEM((LANES, LANES), jnp.int32),
            pltpu.VMEM((CHUNK, N_FEAT), jnp.float32),
            pltpu.VMEM((CHUNK, N_FEAT), jnp.float32),
            pltpu.VMEM((SEG_PER_W, N_FEAT), jnp.float32),
            pltpu.SemaphoreType.DMA,
            pltpu.SemaphoreType.DMA,
            pltpu.SemaphoreType.DMA,
        ],
    )
    return f(X, ids)
